# baseline (speedup 1.0000x reference)
import jax
import jax.numpy as jnp
from jax import lax
from jax.experimental import pallas as pl
from jax.experimental.pallas import tpu as pltpu
from jax.experimental.pallas import tpu_sc as plsc

N_NODES = 100_000
N_EDGES = 3_200_000
F = 16
NC = 2
NS = 16
NW = NC * NS
R2 = 3128
TBL = NW * R2
CHUNK = 4000
NCHUNK = N_EDGES // CHUNK
NVREG = CHUNK // 16
G = 128
SHIFT = 12
MASKO = (1 << SHIFT) - 1


def _sc_body(col_hbm, attr_hbm, out_sums, out_maxs, out_cnts,
             colbuf, idbuf, eid_all, gcolbuf, lrowbuf, rowsbuf, onesbuf,
             zbuf, max_t, sum_t, cnt_sh, csem, gsem):
  c = lax.axis_index("c")
  s = lax.axis_index("s")
  wid = c * NS + s
  node_base = wid * R2
  cnt_base = s * R2
  iota = lax.iota(jnp.int32, 16)
  zeros16 = jnp.zeros((16,), jnp.float32)
  ones16 = jnp.ones((16,), jnp.float32)
  neg16 = jnp.full((16,), -jnp.inf, jnp.float32)
  sent16 = jnp.full((16,), -(1 << SHIFT), jnp.int32) + iota

  @pl.loop(0, G)
  def _(j):
    rowsbuf[0, j, :] = zeros16

  @pl.loop(0, G // 16)
  def _(j):
    onesbuf[pl.ds(j * 16, 16)] = ones16
    zbuf[pl.ds(j * 16, 16)] = zeros16

  @pl.loop(0, R2 + 1)
  def _(r):
    max_t[r, :] = neg16
    sum_t[r, :] = zeros16

  @pl.loop(0, NS * R2 // G)
  def _(i):
    @pl.when(lax.rem(i, NS) == s)
    def _():
      pltpu.sync_copy(zbuf, cnt_sh.at[pl.ds(i * G, G)])

  plsc.subcore_barrier()

  pltpu.async_copy(col_hbm.at[pl.ds(0, CHUNK)], colbuf.at[0], csem)

  @pl.loop(0, NCHUNK)
  def _(k):
    cb = k * CHUNK
    kb = lax.rem(k, 2)
    pltpu.make_async_copy(
        col_hbm.at[pl.ds(cb, CHUNK)], colbuf.at[kb], csem).wait()

    @pl.when(k + 1 < NCHUNK)
    def _():
      pltpu.async_copy(
          col_hbm.at[pl.ds(cb + CHUNK, CHUNK)], colbuf.at[1 - kb], csem)

    def scan_body(j, m):
      cv = colbuf[kb, pl.ds(j * 16, 16)]
      lrow = cv - node_base
      msk = (lrow >= 0) & (lrow < R2)
      pcnt = plsc.all_reduce_population_count(msk)[0]

      @pl.when(pcnt > 0)
      def _():
        packed = (lrow << SHIFT) | (j * 16 + iota)
        pc = plsc.cumsum(msk.astype(jnp.int32))
        plsc.store_scatter(idbuf, [m + pc - 1], packed, mask=msk)

      return m + pcnt

    m = pl.loop(0, NVREG, init_carry=jnp.int32(0), unroll=8)(scan_body)

    for t in range(G // 16):
      idbuf[pl.ds(m + t * 16, 16)] = sent16
    ng = (m + (G - 1)) >> 7

    @pl.loop(0, ng * (G // 16))
    def _(v):
      p = idbuf[pl.ds(v * 16, 16)]
      eid_all[pl.ds(v * 16, 16)] = cb + (p & MASKO)

    @pl.when(ng > 0)
    def _():
      pltpu.async_copy(
          attr_hbm.at[eid_all.at[pl.ds(0, G)]], rowsbuf.at[0], gsem)

    @pl.loop(0, ng)
    def _(g):
      b = lax.rem(g, 2)
      pltpu.make_async_copy(
          attr_hbm.at[eid_all.at[pl.ds(g * G, G)]], rowsbuf.at[b], gsem
      ).wait()

      @pl.when(g + 1 < ng)
      def _():
        pltpu.async_copy(
            attr_hbm.at[eid_all.at[pl.ds(g * G + G, G)]],
            rowsbuf.at[1 - b], gsem)

      for u in range(G // 16):
        p = idbuf[pl.ds(g * G + u * 16, 16)]
        valid = p >= 0
        lrow = p >> SHIFT
        gcolbuf[pl.ds(u * 16, 16)] = jnp.where(valid, lrow + cnt_base, -1)
        lrowbuf[pl.ds(u * 16, 16)] = jnp.where(valid, lrow, R2)
      gidx = plsc.Indices(gcolbuf.at[:], ignored_value=-1)
      pltpu.sync_copy(onesbuf, cnt_sh.at[gidx], add=True)

      qh = jnp.minimum((m - g * G + 15) >> 4, G // 16)

      @pl.loop(0, qh)
      def _(q):
        lrv = lrowbuf[pl.ds(q * 16, 16)]
        for lane in range(16):
          lr = lrv[lane]
          e = q * 16 + lane
          row = rowsbuf[b, e, :]
          max_t[lr, :] = jnp.maximum(max_t[lr, :], row)
          sum_t[lr, :] = sum_t[lr, :] + row

  plsc.subcore_barrier()

  pltpu.sync_copy(sum_t.at[pl.ds(0, R2)], out_sums.at[pl.ds(node_base, R2)])
  pltpu.sync_copy(max_t.at[pl.ds(0, R2)], out_maxs.at[pl.ds(node_base, R2)])
  pltpu.sync_copy(cnt_sh.at[pl.ds(cnt_base, R2)],
                  out_cnts.at[pl.ds(node_base, R2)])


def _sc_aggregate(col, attr):
  mesh = plsc.VectorSubcoreMesh(
      core_axis_name="c", subcore_axis_name="s", num_cores=NC, num_subcores=NS
  )
  kern = pl.kernel(
      _sc_body,
      out_type=(
          jax.ShapeDtypeStruct((TBL, F), jnp.float32),
          jax.ShapeDtypeStruct((TBL, F), jnp.float32),
          jax.ShapeDtypeStruct((TBL,), jnp.float32),
      ),
      mesh=mesh,
      compiler_params=pltpu.CompilerParams(
          needs_layout_passes=False, use_tc_tiling_on_sc=False),
      scratch_types=[
          pltpu.VMEM((2, CHUNK), jnp.int32),
          pltpu.VMEM((CHUNK + 2 * G,), jnp.int32),
          pltpu.VMEM((CHUNK + 2 * G,), jnp.int32),
          pltpu.VMEM((G,), jnp.int32),
          pltpu.VMEM((G,), jnp.int32),
          pltpu.VMEM((2, G, F), jnp.float32),
          pltpu.VMEM((G,), jnp.float32),
          pltpu.VMEM((G,), jnp.float32),
          pltpu.VMEM((R2 + 1, F), jnp.float32),
          pltpu.VMEM((R2 + 1, F), jnp.float32),
          pltpu.VMEM_SHARED((NS * R2,), jnp.float32),
          pltpu.SemaphoreType.DMA,
          pltpu.SemaphoreType.DMA,
      ],
  )
  return kern(col, attr)


BLK = 2000


def _mlp_body(s_ref, mx_ref, c_ref, w1_ref, b1_ref, w2_ref, b2_ref, o_ref):
  ssum = s_ref[...]
  cnt = c_ref[...]
  mx = jnp.where(cnt > 0, mx_ref[...], 0.0)
  mean = ssum / jnp.maximum(cnt, 1.0)
  w1 = w1_ref[...]
  h = (
      jnp.dot(ssum, w1[0:16], preferred_element_type=jnp.float32)
      + jnp.dot(mx, w1[16:32], preferred_element_type=jnp.float32)
      + jnp.dot(mean, w1[32:48], preferred_element_type=jnp.float32)
      + b1_ref[...]
  )
  h = jnp.maximum(h, 0.0)
  o_ref[...] = (
      jnp.dot(h, w2_ref[...], preferred_element_type=jnp.float32) + b2_ref[...]
  )


def _combine_mlp(sums, maxs, cnts, w1, b1, w2, b2):
  return pl.pallas_call(
      _mlp_body,
      grid=(N_NODES // BLK,),
      in_specs=[
          pl.BlockSpec((BLK, F), lambda i: (i, 0)),
          pl.BlockSpec((BLK, F), lambda i: (i, 0)),
          pl.BlockSpec((BLK, 1), lambda i: (i, 0)),
          pl.BlockSpec((3 * F, 128), lambda i: (0, 0)),
          pl.BlockSpec((1, 128), lambda i: (0, 0)),
          pl.BlockSpec((128, 128), lambda i: (0, 0)),
          pl.BlockSpec((1, 128), lambda i: (0, 0)),
      ],
      out_specs=pl.BlockSpec((BLK, 128), lambda i: (i, 0)),
      out_shape=jax.ShapeDtypeStruct((N_NODES, 128), jnp.float32),
  )(sums, maxs, cnts, w1, b1, w2, b2)


def kernel(x, edge_index, edge_attr, u, batch, W1, b1, W2, b2):
  col = edge_index[1].astype(jnp.int32)
  sums, maxs, cnts = _sc_aggregate(col, edge_attr)
  sums = sums[:N_NODES]
  maxs = maxs[:N_NODES]
  cnts = cnts[:N_NODES].reshape(N_NODES, 1)
  return _combine_mlp(sums, maxs, cnts, W1, b1.reshape(1, -1), W2,
                      b2.reshape(1, -1))


# speedup vs baseline: 1.4502x; 1.4502x over previous
import jax
import jax.numpy as jnp
from jax import lax
from jax.experimental import pallas as pl
from jax.experimental.pallas import tpu as pltpu
from jax.experimental.pallas import tpu_sc as plsc

N_NODES = 100_000
N_EDGES = 3_200_000
F = 16
NC = 2
NS = 16
NW = NC * NS
R2 = 3128
TBL = NW * R2
CHUNK = 5120
NCHUNK = N_EDGES // CHUNK
NVREG = CHUNK // 16
G = 128
SHIFT = 13
MASKO = (1 << SHIFT) - 1


def _sc_body(col_hbm, attr_hbm, out_sums, out_maxs, out_cnts,
             colbuf, idbuf, eid_all, gcolbuf, lrowbuf, rowsbuf, onesbuf,
             zbuf, max_t, sum_t, cnt_sh, csem, gsem):
  c = lax.axis_index("c")
  s = lax.axis_index("s")
  wid = c * NS + s
  node_base = wid * R2
  cnt_base = s * R2
  iota = lax.iota(jnp.int32, 16)
  zeros16 = jnp.zeros((16,), jnp.float32)
  ones16 = jnp.ones((16,), jnp.float32)
  neg16 = jnp.full((16,), -jnp.inf, jnp.float32)
  sent16 = jnp.full((16,), -(1 << SHIFT), jnp.int32) + iota

  @pl.loop(0, G)
  def _(j):
    rowsbuf[0, j, :] = zeros16

  @pl.loop(0, G // 16)
  def _(j):
    onesbuf[pl.ds(j * 16, 16)] = ones16
    zbuf[pl.ds(j * 16, 16)] = zeros16

  @pl.loop(0, R2 + 1)
  def _(r):
    max_t[r, :] = neg16
    sum_t[r, :] = zeros16

  @pl.loop(0, NS * R2 // G)
  def _(i):
    @pl.when(lax.rem(i, NS) == s)
    def _():
      pltpu.sync_copy(zbuf, cnt_sh.at[pl.ds(i * G, G)])

  plsc.subcore_barrier()

  pltpu.async_copy(col_hbm.at[pl.ds(0, CHUNK)], colbuf.at[0], csem)

  @pl.loop(0, NCHUNK)
  def _(k):
    cb = k * CHUNK
    kb = lax.rem(k, 2)
    pltpu.make_async_copy(
        col_hbm.at[pl.ds(cb, CHUNK)], colbuf.at[kb], csem).wait()

    @pl.when(k + 1 < NCHUNK)
    def _():
      pltpu.async_copy(
          col_hbm.at[pl.ds(cb + CHUNK, CHUNK)], colbuf.at[1 - kb], csem)

    def scan_body(j, m):
      cv = colbuf[kb, pl.ds(j * 16, 16)]
      lrow = cv - node_base
      msk = (lrow >= 0) & (lrow < R2)
      packed = (lrow << SHIFT) | (j * 16 + iota)
      pc = plsc.cumsum(msk.astype(jnp.int32))
      plsc.store_scatter(idbuf, [m + pc - 1], packed, mask=msk)
      return m + pc[15]

    m = pl.loop(0, NVREG, init_carry=jnp.int32(0), unroll=8)(scan_body)

    for t in range(G // 16):
      idbuf[pl.ds(m + t * 16, 16)] = sent16
    ng = (m + (G - 1)) >> 7

    @pl.loop(0, ng * (G // 16))
    def _(v):
      p = idbuf[pl.ds(v * 16, 16)]
      eid_all[pl.ds(v * 16, 16)] = cb + (p & MASKO)

    @pl.when(ng > 0)
    def _():
      pltpu.async_copy(
          attr_hbm.at[eid_all.at[pl.ds(0, G)]], rowsbuf.at[0], gsem)

    @pl.loop(0, ng)
    def _(g):
      b = lax.rem(g, 2)
      pltpu.make_async_copy(
          attr_hbm.at[eid_all.at[pl.ds(g * G, G)]], rowsbuf.at[b], gsem
      ).wait()

      @pl.when(g + 1 < ng)
      def _():
        pltpu.async_copy(
            attr_hbm.at[eid_all.at[pl.ds(g * G + G, G)]],
            rowsbuf.at[1 - b], gsem)

      for u in range(G // 16):
        p = idbuf[pl.ds(g * G + u * 16, 16)]
        valid = p >= 0
        lrow = p >> SHIFT
        gcolbuf[pl.ds(u * 16, 16)] = jnp.where(valid, lrow + cnt_base, -1)
        lrowbuf[pl.ds(u * 16, 16)] = jnp.where(valid, lrow, R2)
      gidx = plsc.Indices(gcolbuf.at[:], ignored_value=-1)
      pltpu.sync_copy(onesbuf, cnt_sh.at[gidx], add=True)

      qh = jnp.minimum((m - g * G + 15) >> 4, G // 16)

      @pl.loop(0, qh)
      def _(q):
        lrv = lrowbuf[pl.ds(q * 16, 16)]
        for lane in range(16):
          lr = lrv[lane]
          e = q * 16 + lane
          row = rowsbuf[b, e, :]
          max_t[lr, :] = jnp.maximum(max_t[lr, :], row)
          sum_t[lr, :] = sum_t[lr, :] + row

  plsc.subcore_barrier()

  pltpu.sync_copy(sum_t.at[pl.ds(0, R2)], out_sums.at[pl.ds(node_base, R2)])
  pltpu.sync_copy(max_t.at[pl.ds(0, R2)], out_maxs.at[pl.ds(node_base, R2)])
  pltpu.sync_copy(cnt_sh.at[pl.ds(cnt_base, R2)],
                  out_cnts.at[pl.ds(node_base, R2)])


def _sc_aggregate(col, attr):
  mesh = plsc.VectorSubcoreMesh(
      core_axis_name="c", subcore_axis_name="s", num_cores=NC, num_subcores=NS
  )
  kern = pl.kernel(
      _sc_body,
      out_type=(
          jax.ShapeDtypeStruct((TBL, F), jnp.float32),
          jax.ShapeDtypeStruct((TBL, F), jnp.float32),
          jax.ShapeDtypeStruct((TBL,), jnp.float32),
      ),
      mesh=mesh,
      compiler_params=pltpu.CompilerParams(
          needs_layout_passes=False, use_tc_tiling_on_sc=False),
      scratch_types=[
          pltpu.VMEM((2, CHUNK), jnp.int32),
          pltpu.VMEM((CHUNK + 2 * G,), jnp.int32),
          pltpu.VMEM((CHUNK + 2 * G,), jnp.int32),
          pltpu.VMEM((G,), jnp.int32),
          pltpu.VMEM((G,), jnp.int32),
          pltpu.VMEM((2, G, F), jnp.float32),
          pltpu.VMEM((G,), jnp.float32),
          pltpu.VMEM((G,), jnp.float32),
          pltpu.VMEM((R2 + 1, F), jnp.float32),
          pltpu.VMEM((R2 + 1, F), jnp.float32),
          pltpu.VMEM_SHARED((NS * R2,), jnp.float32),
          pltpu.SemaphoreType.DMA,
          pltpu.SemaphoreType.DMA,
      ],
  )
  return kern(col, attr)


BLK = 2000


def _mlp_body(s_ref, mx_ref, c_ref, w1_ref, b1_ref, w2_ref, b2_ref, o_ref):
  ssum = s_ref[...]
  cnt = c_ref[...]
  mx = jnp.where(cnt > 0, mx_ref[...], 0.0)
  mean = ssum / jnp.maximum(cnt, 1.0)
  w1 = w1_ref[...]
  h = (
      jnp.dot(ssum, w1[0:16], preferred_element_type=jnp.float32)
      + jnp.dot(mx, w1[16:32], preferred_element_type=jnp.float32)
      + jnp.dot(mean, w1[32:48], preferred_element_type=jnp.float32)
      + b1_ref[...]
  )
  h = jnp.maximum(h, 0.0)
  o_ref[...] = (
      jnp.dot(h, w2_ref[...], preferred_element_type=jnp.float32) + b2_ref[...]
  )


def _combine_mlp(sums, maxs, cnts, w1, b1, w2, b2):
  return pl.pallas_call(
      _mlp_body,
      grid=(N_NODES // BLK,),
      in_specs=[
          pl.BlockSpec((BLK, F), lambda i: (i, 0)),
          pl.BlockSpec((BLK, F), lambda i: (i, 0)),
          pl.BlockSpec((BLK, 1), lambda i: (i, 0)),
          pl.BlockSpec((3 * F, 128), lambda i: (0, 0)),
          pl.BlockSpec((1, 128), lambda i: (0, 0)),
          pl.BlockSpec((128, 128), lambda i: (0, 0)),
          pl.BlockSpec((1, 128), lambda i: (0, 0)),
      ],
      out_specs=pl.BlockSpec((BLK, 128), lambda i: (i, 0)),
      out_shape=jax.ShapeDtypeStruct((N_NODES, 128), jnp.float32),
  )(sums, maxs, cnts, w1, b1, w2, b2)


def kernel(x, edge_index, edge_attr, u, batch, W1, b1, W2, b2):
  col = edge_index[1].astype(jnp.int32)
  sums, maxs, cnts = _sc_aggregate(col, edge_attr)
  sums = sums[:N_NODES]
  maxs = maxs[:N_NODES]
  cnts = cnts[:N_NODES].reshape(N_NODES, 1)
  return _combine_mlp(sums, maxs, cnts, W1, b1.reshape(1, -1), W2,
                      b2.reshape(1, -1))


# speedup vs baseline: 2.2916x; 1.5802x over previous
import jax
import jax.numpy as jnp
from jax import lax
from jax.experimental import pallas as pl
from jax.experimental.pallas import tpu as pltpu
from jax.experimental.pallas import tpu_sc as plsc

N_NODES = 100_000
N_EDGES = 3_200_000
F = 16
NC = 2
NS = 16
NW = NC * NS
R2 = 3128
TBL = NW * R2
CHUNK = 5120
NCHUNK = N_EDGES // CHUNK
NVREG = CHUNK // 16
G = 128
SHIFT = 13
MASKO = (1 << SHIFT) - 1


def _sc_body(col_hbm, attr_hbm, out_sums, out_maxs, out_cnts,
             colbuf, idbuf, eid_all, gcolbuf, lrowbuf, rowsbuf, onesbuf,
             zbuf, max_t, sum_t, cnt_sh, csem, gsem):
  c = lax.axis_index("c")
  s = lax.axis_index("s")
  wid = c * NS + s
  node_base = wid * R2
  cnt_base = s * R2
  iota = lax.iota(jnp.int32, 16)
  zeros16 = jnp.zeros((16,), jnp.float32)
  ones16 = jnp.ones((16,), jnp.float32)
  neg16 = jnp.full((16,), -jnp.inf, jnp.float32)
  sent16 = jnp.full((16,), -(1 << SHIFT), jnp.int32) + iota

  @pl.loop(0, G)
  def _(j):
    rowsbuf[0, j, :] = zeros16

  @pl.loop(0, G // 16)
  def _(j):
    onesbuf[pl.ds(j * 16, 16)] = ones16
    zbuf[pl.ds(j * 16, 16)] = zeros16

  @pl.loop(0, R2 + 1)
  def _(r):
    max_t[r, :] = neg16
    sum_t[r, :] = zeros16

  @pl.loop(0, NS * R2 // G)
  def _(i):
    @pl.when(lax.rem(i, NS) == s)
    def _():
      pltpu.sync_copy(zbuf, cnt_sh.at[pl.ds(i * G, G)])

  plsc.subcore_barrier()

  pltpu.async_copy(col_hbm.at[pl.ds(0, CHUNK)], colbuf.at[0], csem)

  @pl.loop(0, NCHUNK)
  def _(k):
    cb = k * CHUNK
    kb = lax.rem(k, 2)
    pltpu.make_async_copy(
        col_hbm.at[pl.ds(cb, CHUNK)], colbuf.at[kb], csem).wait()

    @pl.when(k + 1 < NCHUNK)
    def _():
      pltpu.async_copy(
          col_hbm.at[pl.ds(cb + CHUNK, CHUNK)], colbuf.at[1 - kb], csem)

    def scan_body(j, m):
      cv = colbuf[kb, pl.ds(j * 16, 16)]
      lrow = cv - node_base
      msk = (lrow >= 0) & (lrow < R2)
      packed = (lrow << SHIFT) | (j * 16 + iota)
      pc = plsc.cumsum(msk.astype(jnp.int32))
      plsc.store_scatter(idbuf, [m + pc - 1], packed, mask=msk)
      return m + pc[15]

    m = plsc.parallel_loop(0, NVREG, unroll=8, carry=jnp.int32(0))(scan_body)

    for t in range(G // 16):
      idbuf[pl.ds(m + t * 16, 16)] = sent16
    ng = (m + (G - 1)) >> 7

    @plsc.parallel_loop(0, ng * (G // 16), unroll=4)
    def _(v):
      p = idbuf[pl.ds(v * 16, 16)]
      eid_all[pl.ds(v * 16, 16)] = cb + (p & MASKO)

    @pl.when(ng > 0)
    def _():
      pltpu.async_copy(
          attr_hbm.at[eid_all.at[pl.ds(0, G)]], rowsbuf.at[0], gsem)

    @pl.loop(0, ng)
    def _(g):
      b = lax.rem(g, 2)
      pltpu.make_async_copy(
          attr_hbm.at[eid_all.at[pl.ds(g * G, G)]], rowsbuf.at[b], gsem
      ).wait()

      @pl.when(g + 1 < ng)
      def _():
        pltpu.async_copy(
            attr_hbm.at[eid_all.at[pl.ds(g * G + G, G)]],
            rowsbuf.at[1 - b], gsem)

      for u in range(G // 16):
        p = idbuf[pl.ds(g * G + u * 16, 16)]
        valid = p >= 0
        lrow = p >> SHIFT
        gcolbuf[pl.ds(u * 16, 16)] = jnp.where(valid, lrow + cnt_base, -1)
        lrowbuf[pl.ds(u * 16, 16)] = jnp.where(valid, lrow, R2)
      gidx = plsc.Indices(gcolbuf.at[:], ignored_value=-1)
      pltpu.sync_copy(onesbuf, cnt_sh.at[gidx], add=True)

      qh = jnp.minimum((m - g * G + 15) >> 4, G // 16)

      @pl.loop(0, qh)
      def _(q):
        lrv = lrowbuf[pl.ds(q * 16, 16)]
        for lane in range(16):
          lr = lrv[lane]
          e = q * 16 + lane
          row = rowsbuf[b, e, :]
          max_t[lr, :] = jnp.maximum(max_t[lr, :], row)
          sum_t[lr, :] = sum_t[lr, :] + row

  plsc.subcore_barrier()

  pltpu.sync_copy(sum_t.at[pl.ds(0, R2)], out_sums.at[pl.ds(node_base, R2)])
  pltpu.sync_copy(max_t.at[pl.ds(0, R2)], out_maxs.at[pl.ds(node_base, R2)])
  pltpu.sync_copy(cnt_sh.at[pl.ds(cnt_base, R2)],
                  out_cnts.at[pl.ds(node_base, R2)])


def _sc_aggregate(col, attr):
  mesh = plsc.VectorSubcoreMesh(
      core_axis_name="c", subcore_axis_name="s", num_cores=NC, num_subcores=NS
  )
  kern = pl.kernel(
      _sc_body,
      out_type=(
          jax.ShapeDtypeStruct((TBL, F), jnp.float32),
          jax.ShapeDtypeStruct((TBL, F), jnp.float32),
          jax.ShapeDtypeStruct((TBL,), jnp.float32),
      ),
      mesh=mesh,
      compiler_params=pltpu.CompilerParams(
          needs_layout_passes=False, use_tc_tiling_on_sc=False),
      scratch_types=[
          pltpu.VMEM((2, CHUNK), jnp.int32),
          pltpu.VMEM((CHUNK + 2 * G,), jnp.int32),
          pltpu.VMEM((CHUNK + 2 * G,), jnp.int32),
          pltpu.VMEM((G,), jnp.int32),
          pltpu.VMEM((G,), jnp.int32),
          pltpu.VMEM((2, G, F), jnp.float32),
          pltpu.VMEM((G,), jnp.float32),
          pltpu.VMEM((G,), jnp.float32),
          pltpu.VMEM((R2 + 1, F), jnp.float32),
          pltpu.VMEM((R2 + 1, F), jnp.float32),
          pltpu.VMEM_SHARED((NS * R2,), jnp.float32),
          pltpu.SemaphoreType.DMA,
          pltpu.SemaphoreType.DMA,
      ],
  )
  return kern(col, attr)


BLK = 2000


def _mlp_body(s_ref, mx_ref, c_ref, w1_ref, b1_ref, w2_ref, b2_ref, o_ref):
  ssum = s_ref[...]
  cnt = c_ref[...]
  mx = jnp.where(cnt > 0, mx_ref[...], 0.0)
  mean = ssum / jnp.maximum(cnt, 1.0)
  w1 = w1_ref[...]
  h = (
      jnp.dot(ssum, w1[0:16], preferred_element_type=jnp.float32)
      + jnp.dot(mx, w1[16:32], preferred_element_type=jnp.float32)
      + jnp.dot(mean, w1[32:48], preferred_element_type=jnp.float32)
      + b1_ref[...]
  )
  h = jnp.maximum(h, 0.0)
  o_ref[...] = (
      jnp.dot(h, w2_ref[...], preferred_element_type=jnp.float32) + b2_ref[...]
  )


def _combine_mlp(sums, maxs, cnts, w1, b1, w2, b2):
  return pl.pallas_call(
      _mlp_body,
      grid=(N_NODES // BLK,),
      in_specs=[
          pl.BlockSpec((BLK, F), lambda i: (i, 0)),
          pl.BlockSpec((BLK, F), lambda i: (i, 0)),
          pl.BlockSpec((BLK, 1), lambda i: (i, 0)),
          pl.BlockSpec((3 * F, 128), lambda i: (0, 0)),
          pl.BlockSpec((1, 128), lambda i: (0, 0)),
          pl.BlockSpec((128, 128), lambda i: (0, 0)),
          pl.BlockSpec((1, 128), lambda i: (0, 0)),
      ],
      out_specs=pl.BlockSpec((BLK, 128), lambda i: (i, 0)),
      out_shape=jax.ShapeDtypeStruct((N_NODES, 128), jnp.float32),
  )(sums, maxs, cnts, w1, b1, w2, b2)


def kernel(x, edge_index, edge_attr, u, batch, W1, b1, W2, b2):
  col = edge_index[1].astype(jnp.int32)
  sums, maxs, cnts = _sc_aggregate(col, edge_attr)
  sums = sums[:N_NODES]
  maxs = maxs[:N_NODES]
  cnts = cnts[:N_NODES].reshape(N_NODES, 1)
  return _combine_mlp(sums, maxs, cnts, W1, b1.reshape(1, -1), W2,
                      b2.reshape(1, -1))


# speedup vs baseline: 2.3324x; 1.0178x over previous
import jax
import jax.numpy as jnp
from jax import lax
from jax.experimental import pallas as pl
from jax.experimental.pallas import tpu as pltpu
from jax.experimental.pallas import tpu_sc as plsc

N_NODES = 100_000
N_EDGES = 3_200_000
F = 16
NC = 2
NS = 16
NW = NC * NS
R2 = 3128
TBL = NW * R2
CHUNK = 5120
NCHUNK = N_EDGES // CHUNK
NVREG = CHUNK // 16
G = 128
SHIFT = 13
MASKO = (1 << SHIFT) - 1


def _sc_body(col_hbm, attr_hbm, out_sums, out_maxs, out_cnts,
             colbuf, idbuf, eid_all, gcolbuf, lrowbuf, rowsbuf, onesbuf,
             zbuf, max_t, sum_t, cnt_sh, csem, gsem, ssem):
  c = lax.axis_index("c")
  s = lax.axis_index("s")
  wid = c * NS + s
  node_base = wid * R2
  cnt_base = s * R2
  iota = lax.iota(jnp.int32, 16)
  zeros16 = jnp.zeros((16,), jnp.float32)
  ones16 = jnp.ones((16,), jnp.float32)
  neg16 = jnp.full((16,), -jnp.inf, jnp.float32)
  sent16 = jnp.full((16,), -(1 << SHIFT), jnp.int32) + iota

  @pl.loop(0, G)
  def _(j):
    rowsbuf[0, j, :] = zeros16

  @pl.loop(0, G // 16)
  def _(j):
    onesbuf[pl.ds(j * 16, 16)] = ones16
    zbuf[pl.ds(j * 16, 16)] = zeros16

  @pl.loop(0, R2 + 1)
  def _(r):
    max_t[r, :] = neg16
    sum_t[r, :] = zeros16

  @pl.loop(0, NS * R2 // G)
  def _(i):
    @pl.when(lax.rem(i, NS) == s)
    def _():
      pltpu.sync_copy(zbuf, cnt_sh.at[pl.ds(i * G, G)])

  plsc.subcore_barrier()

  pltpu.async_copy(col_hbm.at[pl.ds(0, CHUNK)], colbuf.at[0], csem)

  @pl.loop(0, NCHUNK)
  def _(k):
    cb = k * CHUNK
    kb = lax.rem(k, 2)
    pltpu.make_async_copy(
        col_hbm.at[pl.ds(cb, CHUNK)], colbuf.at[kb], csem).wait()

    @pl.when(k + 1 < NCHUNK)
    def _():
      pltpu.async_copy(
          col_hbm.at[pl.ds(cb + CHUNK, CHUNK)], colbuf.at[1 - kb], csem)

    def scan_body(j, m):
      cv = colbuf[kb, pl.ds(j * 16, 16)]
      lrow = cv - node_base
      msk = (lrow >= 0) & (lrow < R2)
      packed = (lrow << SHIFT) | (j * 16 + iota)
      pc = plsc.cumsum(msk.astype(jnp.int32))
      plsc.store_scatter(idbuf, [m + pc - 1], packed, mask=msk)
      return m + pc[15]

    m = plsc.parallel_loop(0, NVREG, unroll=16, carry=jnp.int32(0))(scan_body)

    for t in range(G // 16):
      idbuf[pl.ds(m + t * 16, 16)] = sent16
    ng = (m + (G - 1)) >> 7

    @plsc.parallel_loop(0, ng * (G // 16), unroll=4)
    def _(v):
      p = idbuf[pl.ds(v * 16, 16)]
      eid_all[pl.ds(v * 16, 16)] = cb + (p & MASKO)

    @pl.when(ng > 0)
    def _():
      pltpu.async_copy(
          attr_hbm.at[eid_all.at[pl.ds(0, G)]], rowsbuf.at[0], gsem)

    @pl.loop(0, ng)
    def _(g):
      b = lax.rem(g, 2)
      pltpu.make_async_copy(
          attr_hbm.at[eid_all.at[pl.ds(g * G, G)]], rowsbuf.at[b], gsem
      ).wait()

      @pl.when(g + 1 < ng)
      def _():
        pltpu.async_copy(
            attr_hbm.at[eid_all.at[pl.ds(g * G + G, G)]],
            rowsbuf.at[1 - b], gsem)

      for u in range(G // 16):
        p = idbuf[pl.ds(g * G + u * 16, 16)]
        valid = p >= 0
        lrow = p >> SHIFT
        gcolbuf[pl.ds(u * 16, 16)] = jnp.where(valid, lrow + cnt_base, -1)
        lrowbuf[pl.ds(u * 16, 16)] = jnp.where(valid, lrow, R2)
      gidx = plsc.Indices(gcolbuf.at[:], ignored_value=-1)
      cnt_dma = pltpu.async_copy(onesbuf, cnt_sh.at[gidx], ssem, add=True)

      qh = jnp.minimum((m - g * G + 15) >> 4, G // 16)

      @pl.loop(0, qh)
      def _(q):
        lrv = lrowbuf[pl.ds(q * 16, 16)]
        for lane in range(16):
          lr = lrv[lane]
          e = q * 16 + lane
          row = rowsbuf[b, e, :]
          max_t[lr, :] = jnp.maximum(max_t[lr, :], row)
          sum_t[lr, :] = sum_t[lr, :] + row

      cnt_dma.wait()

  plsc.subcore_barrier()

  pltpu.sync_copy(sum_t.at[pl.ds(0, R2)], out_sums.at[pl.ds(node_base, R2)])
  pltpu.sync_copy(max_t.at[pl.ds(0, R2)], out_maxs.at[pl.ds(node_base, R2)])
  pltpu.sync_copy(cnt_sh.at[pl.ds(cnt_base, R2)],
                  out_cnts.at[pl.ds(node_base, R2)])


def _sc_aggregate(col, attr):
  mesh = plsc.VectorSubcoreMesh(
      core_axis_name="c", subcore_axis_name="s", num_cores=NC, num_subcores=NS
  )
  kern = pl.kernel(
      _sc_body,
      out_type=(
          jax.ShapeDtypeStruct((TBL, F), jnp.float32),
          jax.ShapeDtypeStruct((TBL, F), jnp.float32),
          jax.ShapeDtypeStruct((TBL,), jnp.float32),
      ),
      mesh=mesh,
      compiler_params=pltpu.CompilerParams(
          needs_layout_passes=False, use_tc_tiling_on_sc=False),
      scratch_types=[
          pltpu.VMEM((2, CHUNK), jnp.int32),
          pltpu.VMEM((CHUNK + 2 * G,), jnp.int32),
          pltpu.VMEM((CHUNK + 2 * G,), jnp.int32),
          pltpu.VMEM((G,), jnp.int32),
          pltpu.VMEM((G,), jnp.int32),
          pltpu.VMEM((2, G, F), jnp.float32),
          pltpu.VMEM((G,), jnp.float32),
          pltpu.VMEM((G,), jnp.float32),
          pltpu.VMEM((R2 + 1, F), jnp.float32),
          pltpu.VMEM((R2 + 1, F), jnp.float32),
          pltpu.VMEM_SHARED((NS * R2,), jnp.float32),
          pltpu.SemaphoreType.DMA,
          pltpu.SemaphoreType.DMA,
          pltpu.SemaphoreType.DMA,
      ],
  )
  return kern(col, attr)


BLK = 2000


def _mlp_body(s_ref, mx_ref, c_ref, w1_ref, b1_ref, w2_ref, b2_ref, o_ref):
  ssum = s_ref[...]
  cnt = c_ref[...]
  mx = jnp.where(cnt > 0, mx_ref[...], 0.0)
  mean = ssum / jnp.maximum(cnt, 1.0)
  w1 = w1_ref[...]
  h = (
      jnp.dot(ssum, w1[0:16], preferred_element_type=jnp.float32)
      + jnp.dot(mx, w1[16:32], preferred_element_type=jnp.float32)
      + jnp.dot(mean, w1[32:48], preferred_element_type=jnp.float32)
      + b1_ref[...]
  )
  h = jnp.maximum(h, 0.0)
  o_ref[...] = (
      jnp.dot(h, w2_ref[...], preferred_element_type=jnp.float32) + b2_ref[...]
  )


def _combine_mlp(sums, maxs, cnts, w1, b1, w2, b2):
  return pl.pallas_call(
      _mlp_body,
      grid=(N_NODES // BLK,),
      in_specs=[
          pl.BlockSpec((BLK, F), lambda i: (i, 0)),
          pl.BlockSpec((BLK, F), lambda i: (i, 0)),
          pl.BlockSpec((BLK, 1), lambda i: (i, 0)),
          pl.BlockSpec((3 * F, 128), lambda i: (0, 0)),
          pl.BlockSpec((1, 128), lambda i: (0, 0)),
          pl.BlockSpec((128, 128), lambda i: (0, 0)),
          pl.BlockSpec((1, 128), lambda i: (0, 0)),
      ],
      out_specs=pl.BlockSpec((BLK, 128), lambda i: (i, 0)),
      out_shape=jax.ShapeDtypeStruct((N_NODES, 128), jnp.float32),
  )(sums, maxs, cnts, w1, b1, w2, b2)


def kernel(x, edge_index, edge_attr, u, batch, W1, b1, W2, b2):
  col = edge_index[1].astype(jnp.int32)
  sums, maxs, cnts = _sc_aggregate(col, edge_attr)
  sums = sums[:N_NODES]
  maxs = maxs[:N_NODES]
  cnts = cnts[:N_NODES].reshape(N_NODES, 1)
  return _combine_mlp(sums, maxs, cnts, W1, b1.reshape(1, -1), W2,
                      b2.reshape(1, -1))
